# Initial kernel scaffold; baseline (speedup 1.0000x reference)
import jax
import jax.numpy as jnp
from jax.experimental import pallas as pl


def kernel(x, sW1_0, sb1_0, sW2_0, sb2_0, sW1_1, sb1_1, sW2_1, sb2_1, tW1_0, tb1_0, tW2_0, tb2_0, tW1_1, tb1_1, tW2_1, tb2_1, Wc1, bc1, Wc2, bc2):
    raise NotImplementedError("write your pallas kernel here")


# speedup vs baseline: 4.1007x; 4.1007x over previous
import numpy as np
import jax
import jax.numpy as jnp
from jax.experimental import pallas as pl

_B, _N, _T, _F = 8, 2048, 96, 16
_TO, _E, _K = 12, 2, 25
_NB = 256


def _avg_mat(t, k):
    p = (k - 1) // 2
    a = np.zeros((t, t), np.float32)
    for to in range(t):
        for j in range(to - p, to + p + 1):
            a[min(max(j, 0), t - 1), to] += 1.0 / k
    return a


def _down_mat(t):
    d = np.zeros((t, t // 2), np.float32)
    for i in range(t // 2):
        d[2 * i, i] = 0.5
        d[2 * i + 1, i] = 0.5
    return d


_A96 = _avg_mat(96, _K)
_A48 = _avg_mat(48, _K)
_A24 = _avg_mat(24, _K)
_D96 = _down_mat(96)
_D48 = _down_mat(48)
_INV_SQRT2 = np.float32(1.0 / np.sqrt(2.0))


def _gelu(v):
    return 0.5 * v * (1.0 + jax.lax.erf(v * _INV_SQRT2))


def _body(x_ref, a96_ref, a48_ref, a24_ref, d96_ref, d48_ref,
          sw10_ref, sb10_ref, sw20_ref, sb20_ref,
          sw11_ref, sb11_ref, sw21_ref, sb21_ref,
          tw10_ref, tb10_ref, tw20_ref, tb20_ref,
          tw11_ref, tb11_ref, tw21_ref, tb21_ref,
          wc1_ref, bc1_ref, wc2_ref, wc2b_ref, bc2_ref, out_ref):
    nb = x_ref.shape[0]
    m = nb * _F
    xt = x_ref[...]
    x0 = jnp.swapaxes(xt, 1, 2).reshape(m, _T)
    x1 = jnp.dot(x0, d96_ref[...])
    x2 = jnp.dot(x1, d48_ref[...])
    for b in range(_E):
        m0 = jnp.dot(x0, a96_ref[...])
        m1 = jnp.dot(x1, a48_ref[...])
        m2 = jnp.dot(x2, a24_ref[...])
        s0 = x0 - m0
        s1 = x1 - m1
        s2 = x2 - m2
        g = _gelu(jnp.dot(s0, sw10_ref[b]) + sb10_ref[b])
        sb1 = jnp.dot(g, sw20_ref[b]) + sb20_ref[b] + s1
        g = _gelu(jnp.dot(sb1, sw11_ref[b]) + sb11_ref[b])
        sb2 = jnp.dot(g, sw21_ref[b]) + sb21_ref[b] + s2
        g = _gelu(jnp.dot(m2, tw11_ref[b]) + tb11_ref[b])
        tt1 = jnp.dot(g, tw21_ref[b]) + tb21_ref[b] + m1
        g = _gelu(jnp.dot(tt1, tw10_ref[b]) + tb10_ref[b])
        tt0 = jnp.dot(g, tw20_ref[b]) + tb20_ref[b] + m0
        x0 = s0 + tt0
        x1 = sb1 + tt1
        x2 = sb2 + m2
    z = jnp.sum(x0.reshape(nb, _F, _T) * wc2b_ref[...], axis=1)
    bhead = bc1_ref[...] * jnp.sum(wc2_ref[...]) + bc2_ref[0, 0]
    out_ref[...] = jnp.dot(z, wc1_ref[...]) + bhead


def kernel(x, sW1_0, sb1_0, sW2_0, sb2_0, sW1_1, sb1_1, sW2_1, sb2_1,
           tW1_0, tb1_0, tW2_0, tb2_0, tW1_1, tb1_1, tW2_1, tb2_1,
           Wc1, bc1, Wc2, bc2):
    xf = x.reshape(_B * _N, _T, _F)
    tr = lambda w: jnp.swapaxes(w, 1, 2)
    bi = lambda v: v.reshape(_E, 1, -1)
    ops = (
        jnp.asarray(_A96), jnp.asarray(_A48), jnp.asarray(_A24),
        jnp.asarray(_D96), jnp.asarray(_D48),
        tr(sW1_0), bi(sb1_0), tr(sW2_0), bi(sb2_0),
        tr(sW1_1), bi(sb1_1), tr(sW2_1), bi(sb2_1),
        tr(tW1_0), bi(tb1_0), tr(tW2_0), bi(tb2_0),
        tr(tW1_1), bi(tb1_1), tr(tW2_1), bi(tb2_1),
        Wc1.T, bc1.reshape(1, _TO), Wc2,
        jnp.broadcast_to(Wc2.reshape(1, _F, 1), (1, _F, _T)),
        bc2.reshape(1, 1),
    )
    full = lambda a: pl.BlockSpec(a.shape, lambda i: (0,) * a.ndim)
    grid = (_B * _N // _NB,)
    out = pl.pallas_call(
        _body,
        grid=grid,
        in_specs=[pl.BlockSpec((_NB, _T, _F), lambda i: (i, 0, 0))]
                 + [full(a) for a in ops],
        out_specs=pl.BlockSpec((_NB, _TO), lambda i: (i, 0)),
        out_shape=jax.ShapeDtypeStruct((_B * _N, _TO), jnp.float32),
    )(xf, *ops)
    return out.reshape(_B, _N, _TO)
